# baseline (speedup 1.0000x reference)
import functools

import jax
import jax.numpy as jnp
import numpy as np
from jax import lax
from jax.experimental import pallas as pl
from jax.experimental.pallas import tpu as pltpu
from jax.experimental.pallas import tpu_sc as plsc

_N = 20000
_C = 10
_NCLS = _C - 1
_SCORE_THRESH = 0.05
_NMS_THRESH = 0.5
_DET = 100
_TOP = 512
_CLIP = float(np.log(1000.0 / 16.0))
_W = _TOP // 32


def _nms_sc_body(sup_hbm, valid_hbm, out_hbm, sup_v, keep_v):
    nc = 2
    wid = lax.axis_index("s") * nc + lax.axis_index("c")

    @pl.when(wid < _NCLS)
    def _():
        pltpu.sync_copy(sup_hbm.at[wid], sup_v)
        pltpu.sync_copy(valid_hbm.at[wid], keep_v)

        dnums = lax.GatherDimensionNumbers(
            offset_dims=(), collapsed_slice_dims=(0,), start_index_map=(0,)
        )

        def body(i, keep):
            w = lax.shift_right_logical(i, 5)
            b = lax.bitwise_and(i, 31)
            w_vec = jnp.full((16,), w, jnp.int32)
            word = lax.gather(
                keep,
                w_vec[:, None],
                dimension_numbers=dnums,
                slice_sizes=(1,),
                mode=lax.GatherScatterMode.PROMISE_IN_BOUNDS,
            )
            b_vec = jnp.full((16,), b, jnp.int32)
            bit = lax.bitwise_and(lax.shift_right_logical(word, b_vec), 1)
            mask = jnp.where(bit == 1, -1, 0).astype(jnp.int32)
            row = sup_v[pl.ds(i * _W, _W)]
            return lax.bitwise_and(
                keep, lax.bitwise_not(lax.bitwise_and(row, mask))
            )

        keep_v[:] = lax.fori_loop(0, _TOP, body, keep_v[:])
        pltpu.sync_copy(keep_v, out_hbm.at[wid])


@jax.jit
def _run_nms(sup_words, valid_words):
    mesh = plsc.VectorSubcoreMesh(core_axis_name="c", subcore_axis_name="s")
    f = pl.kernel(
        _nms_sc_body,
        out_type=jax.ShapeDtypeStruct((_NCLS, _W), jnp.int32),
        scratch_types=[
            pltpu.VMEM((_TOP * _W,), jnp.int32),
            pltpu.VMEM((_W,), jnp.int32),
        ],
        mesh=mesh,
    )
    return f(sup_words, valid_words)


def _cmpex(v, ix, j, up, ivec):
    left = (ivec & j) == 0
    pvp = jnp.concatenate([v[:, j:], v[:, :j]], -1)
    pvm = jnp.concatenate([v[:, -j:], v[:, :-j]], -1)
    pip = jnp.concatenate([ix[:, j:], ix[:, :j]], -1)
    pim = jnp.concatenate([ix[:, -j:], ix[:, :-j]], -1)
    pv = jnp.where(left, pvp, pvm)
    pi = jnp.where(left, pip, pim)
    a_first = (v > pv) | ((v == pv) & (ix < pi))
    keep = a_first ^ up ^ left
    return jnp.where(keep, v, pv), jnp.where(keep, ix, pi)


def _bitonic_topk(v, ix, ch, k_out):
    bsz, width = v.shape

    def iota(w):
        return lax.broadcasted_iota(jnp.int32, (bsz, w), 1)

    ivec = iota(width)
    k = 2
    while k <= ch:
        j = k // 2
        while j >= 1:
            v, ix = _cmpex(v, ix, j, (ivec & k) == 0, ivec)
            j //= 2
        k *= 2
    while width > ch:
        nl = width // ch
        if nl % 2 == 1:
            v = jnp.concatenate(
                [v, jnp.full((bsz, ch), -3.0, v.dtype)], -1
            )
            ix = jnp.concatenate(
                [ix, jnp.full((bsz, ch), jnp.int32(2**30)), ], -1
            )
            width += ch
            nl += 1
        ivec = iota(width)
        j = ch
        while j >= 1:
            v, ix = _cmpex(v, ix, j, (ivec & (2 * ch)) == 0, ivec)
            j //= 2
        offs = [b * 2 * ch + (0 if b % 2 == 0 else ch) for b in range(nl // 2)]
        v = jnp.concatenate([v[:, o : o + ch] for o in offs], -1)
        ix = jnp.concatenate([ix[:, o : o + ch] for o in offs], -1)
        width //= 2
        ivec = iota(width)
    return v[:, :k_out], ix[:, :k_out]


_PADW = 20480


def _topk_body(logits_ref, tops_ref, topi_ref):
    x = logits_ref[...]
    m = jnp.max(x, axis=0, keepdims=True)
    e = jnp.exp(x - m)
    denom = jnp.sum(e, axis=0, keepdims=True)
    s = e[1:, :] / denom
    sm = jnp.where(s > _SCORE_THRESH, s, -1.0)
    v = jnp.concatenate(
        [sm, jnp.full((_NCLS, _PADW - _N), -2.0, jnp.float32)], -1
    )
    ix = lax.broadcasted_iota(jnp.int32, (_NCLS, _PADW), 1)
    tv, ti = _bitonic_topk(v, ix, _TOP, _TOP)
    tops_ref[...] = tv
    topi_ref[...] = ti


@jax.jit
def _topk512(logits_t):
    return pl.pallas_call(
        _topk_body,
        out_shape=[
            jax.ShapeDtypeStruct((_NCLS, _TOP), jnp.float32),
            jax.ShapeDtypeStruct((_NCLS, _TOP), jnp.int32),
        ],
    )(logits_t)


_FINW = 8192


def _final_body(s_ref, fs_ref, fi_ref):
    s = s_ref[...]
    v = jnp.concatenate(
        [s, jnp.full((1, _FINW - _NCLS * _TOP), -2.0, jnp.float32)], -1
    )
    ix = lax.broadcasted_iota(jnp.int32, (1, _FINW), 1)
    tv, ti = _bitonic_topk(v, ix, _TOP, _DET)
    fs_ref[...] = tv
    fi_ref[...] = ti


@jax.jit
def _final_topk(scores_cat):
    return pl.pallas_call(
        _final_body,
        out_shape=[
            jax.ShapeDtypeStruct((1, _DET), jnp.float32),
            jax.ShapeDtypeStruct((1, _DET), jnp.int32),
        ],
    )(scores_cat.reshape(1, _NCLS * _TOP))


def _iou_pack_body(props_ref, reg_ref, tops_ref, boxes_ref, sup_ref, valid_ref):
    props = props_ref[0]
    reg = reg_ref[0]
    ctr = props[:, 0:3]
    size = jnp.abs(props[:, 3:6]) + 1e-3
    theta = props[:, 6:7]
    pred_ctr = (reg[:, 0:3] / 10.0) * size + ctr
    pred_size = jnp.exp(jnp.minimum(reg[:, 3:6] / 5.0, _CLIP)) * size
    pred_theta = theta + reg[:, 6:7]
    boxes = jnp.concatenate(
        [pred_ctr, pred_size, pred_theta, jnp.zeros((_TOP, 1), jnp.float32)],
        axis=-1,
    )
    boxes_ref[0] = boxes

    x1 = boxes[:, 0] - boxes[:, 3] * 0.5
    x2 = boxes[:, 0] + boxes[:, 3] * 0.5
    y1 = boxes[:, 1] - boxes[:, 4] * 0.5
    y2 = boxes[:, 1] + boxes[:, 4] * 0.5
    z1 = boxes[:, 2]
    z2 = boxes[:, 2] + boxes[:, 5]

    def inter(a1, a2):
        lo = jnp.maximum(a1[:, None], a1[None, :])
        hi = jnp.minimum(a2[:, None], a2[None, :])
        return jnp.clip(hi - lo, 0.0)

    iv = inter(x1, x2) * inter(y1, y2) * inter(z1, z2)
    vol = (
        jnp.clip(x2 - x1, 0.0)
        * jnp.clip(y2 - y1, 0.0)
        * jnp.clip(z2 - z1, 0.0)
    )
    union = vol[:, None] + vol[None, :] - iv
    iou = iv / jnp.maximum(union, 1e-8)
    col = lax.broadcasted_iota(jnp.int32, (_TOP, _TOP), 1)
    row = lax.broadcasted_iota(jnp.int32, (_TOP, _TOP), 0)
    sup = ((iou > _NMS_THRESH) & (col > row)).astype(jnp.int32)
    weights = jnp.left_shift(
        jnp.int32(1), jnp.arange(32, dtype=jnp.int32)
    )
    words = [
        jnp.sum(sup[:, 32 * w : 32 * w + 32] * weights[None, :], axis=-1)
        for w in range(_W)
    ]
    sup_ref[0] = jnp.concatenate([x[:, None] for x in words], axis=-1)

    vbits = (tops_ref[0, 0] > _SCORE_THRESH).astype(jnp.int32)
    vwords = [
        jnp.sum(vbits[32 * w : 32 * w + 32] * weights) for w in range(_W)
    ]
    valid_ref[0, 0] = jnp.stack(vwords)


@jax.jit
def _iou_pack(props, reg, top_s):
    return pl.pallas_call(
        _iou_pack_body,
        grid=(_NCLS,),
        in_specs=[
            pl.BlockSpec((1, _TOP, 8), lambda c: (c, 0, 0)),
            pl.BlockSpec((1, _TOP, 8), lambda c: (c, 0, 0)),
            pl.BlockSpec((1, 1, _TOP), lambda c: (c, 0, 0)),
        ],
        out_specs=[
            pl.BlockSpec((1, _TOP, 8), lambda c: (c, 0, 0)),
            pl.BlockSpec((1, _TOP, _W), lambda c: (c, 0, 0)),
            pl.BlockSpec((1, 1, _W), lambda c: (c, 0, 0)),
        ],
        out_shape=[
            jax.ShapeDtypeStruct((_NCLS, _TOP, 8), jnp.float32),
            jax.ShapeDtypeStruct((_NCLS, _TOP, _W), jnp.int32),
            jax.ShapeDtypeStruct((_NCLS, 1, _W), jnp.int32),
        ],
    )(props, reg, top_s.reshape(_NCLS, 1, _TOP))


def _pack_bits(bits):
    shape = bits.shape[:-1] + (bits.shape[-1] // 32, 32)
    weights = jnp.left_shift(
        jnp.uint32(1), jnp.arange(32, dtype=jnp.uint32)
    )
    words = jnp.sum(bits.reshape(shape).astype(jnp.uint32) * weights, axis=-1)
    return lax.bitcast_convert_type(words, jnp.int32)


def kernel(class_logits, box_regression, corners_semantic, proposals):
    top_s, top_i = _topk512(class_logits.T)

    props = jnp.pad(proposals, ((0, 0), (0, 1)))[top_i]
    reg_all = jnp.pad(
        box_regression.reshape(_N, _C, 7), ((0, 0), (0, 0), (0, 1))
    )
    cls_idx = jnp.arange(1, _C, dtype=jnp.int32)[:, None]
    reg = reg_all[top_i, cls_idx]

    boxes8, sup_words3, valid_words3 = _iou_pack(props, reg, top_s)
    sup_words = sup_words3.reshape(_NCLS, _TOP * _W)
    valid_words = valid_words3.reshape(_NCLS, _W)

    keep_words = _run_nms(sup_words, valid_words)
    keep = (
        jnp.right_shift(
            lax.bitcast_convert_type(keep_words, jnp.uint32)[:, :, None],
            jnp.arange(32, dtype=jnp.uint32)[None, None, :],
        )
        & 1
    ).astype(bool).reshape(_NCLS, _TOP)

    s_final = jnp.where(keep, top_s, -1.0)
    fs, fi = _final_topk(s_final)
    final_s = fs[0]
    final_idx = fi[0]
    boxes_cat = boxes8.reshape(-1, 8)
    final_b = boxes_cat[final_idx][:, :7]
    final_l = (final_idx // _TOP + 1).astype(jnp.int32)
    return final_b, final_s, final_l


# speedup vs baseline: 1.0047x; 1.0047x over previous
import functools

import jax
import jax.numpy as jnp
import numpy as np
from jax import lax
from jax.experimental import pallas as pl
from jax.experimental.pallas import tpu as pltpu
from jax.experimental.pallas import tpu_sc as plsc

_N = 20000
_C = 10
_NCLS = _C - 1
_SCORE_THRESH = 0.05
_NMS_THRESH = 0.5
_DET = 100
_TOP = 512
_CLIP = float(np.log(1000.0 / 16.0))
_W = _TOP // 32


def _nms_sc_body(sup_hbm, valid_hbm, out_hbm, sup_v, keep_v):
    nc = 2
    wid = lax.axis_index("s") * nc + lax.axis_index("c")

    @pl.when(wid < _NCLS)
    def _():
        pltpu.sync_copy(sup_hbm.at[wid], sup_v)
        pltpu.sync_copy(valid_hbm.at[wid], keep_v)

        dnums = lax.GatherDimensionNumbers(
            offset_dims=(), collapsed_slice_dims=(0,), start_index_map=(0,)
        )

        def body(i, keep):
            w = lax.shift_right_logical(i, 5)
            b = lax.bitwise_and(i, 31)
            w_vec = jnp.full((16,), w, jnp.int32)
            word = lax.gather(
                keep,
                w_vec[:, None],
                dimension_numbers=dnums,
                slice_sizes=(1,),
                mode=lax.GatherScatterMode.PROMISE_IN_BOUNDS,
            )
            b_vec = jnp.full((16,), b, jnp.int32)
            bit = lax.bitwise_and(lax.shift_right_logical(word, b_vec), 1)
            mask = jnp.where(bit == 1, -1, 0).astype(jnp.int32)
            row = sup_v[pl.ds(i * _W, _W)]
            return lax.bitwise_and(
                keep, lax.bitwise_not(lax.bitwise_and(row, mask))
            )

        keep_v[:] = lax.fori_loop(0, _TOP, body, keep_v[:])
        pltpu.sync_copy(keep_v, out_hbm.at[wid])


@jax.jit
def _run_nms(sup_words, valid_words):
    mesh = plsc.VectorSubcoreMesh(core_axis_name="c", subcore_axis_name="s")
    f = pl.kernel(
        _nms_sc_body,
        out_type=jax.ShapeDtypeStruct((_NCLS, _W), jnp.int32),
        scratch_types=[
            pltpu.VMEM((_TOP * _W,), jnp.int32),
            pltpu.VMEM((_W,), jnp.int32),
        ],
        mesh=mesh,
    )
    return f(sup_words, valid_words)


def _cmpex(v, ix, j, up, ivec):
    left = (ivec & j) == 0
    pvp = jnp.concatenate([v[:, j:], v[:, :j]], -1)
    pvm = jnp.concatenate([v[:, -j:], v[:, :-j]], -1)
    pip = jnp.concatenate([ix[:, j:], ix[:, :j]], -1)
    pim = jnp.concatenate([ix[:, -j:], ix[:, :-j]], -1)
    pv = jnp.where(left, pvp, pvm)
    pi = jnp.where(left, pip, pim)
    a_first = (v > pv) | ((v == pv) & (ix < pi))
    keep = a_first ^ up ^ left
    return jnp.where(keep, v, pv), jnp.where(keep, ix, pi)


def _bitonic_topk(v, ix, ch, k_out):
    bsz, width = v.shape

    def iota(w):
        return lax.broadcasted_iota(jnp.int32, (bsz, w), 1)

    ivec = iota(width)
    k = 2
    while k <= ch:
        j = k // 2
        while j >= 1:
            v, ix = _cmpex(v, ix, j, (ivec & k) == 0, ivec)
            j //= 2
        k *= 2
    while width > ch:
        nl = width // ch
        if nl % 2 == 1:
            v = jnp.concatenate(
                [v, jnp.full((bsz, ch), -3.0, v.dtype)], -1
            )
            ix = jnp.concatenate(
                [ix, jnp.full((bsz, ch), jnp.int32(2**30)), ], -1
            )
            width += ch
            nl += 1
        ivec = iota(width)
        j = ch
        while j >= 1:
            v, ix = _cmpex(v, ix, j, (ivec & (2 * ch)) == 0, ivec)
            j //= 2
        offs = [b * 2 * ch + (0 if b % 2 == 0 else ch) for b in range(nl // 2)]
        v = jnp.concatenate([v[:, o : o + ch] for o in offs], -1)
        ix = jnp.concatenate([ix[:, o : o + ch] for o in offs], -1)
        width //= 2
        ivec = iota(width)
    return v[:, :k_out], ix[:, :k_out]


_PADW = 20480


def _topk_body(logits_ref, tops_ref, topi_ref):
    x = logits_ref[...]
    m = jnp.max(x, axis=0, keepdims=True)
    e = jnp.exp(x - m)
    denom = jnp.sum(e, axis=0, keepdims=True)
    s = e[1:, :] / denom
    sm = jnp.where(s > _SCORE_THRESH, s, -1.0)
    v = jnp.concatenate(
        [sm, jnp.full((_NCLS, _PADW - _N), -2.0, jnp.float32)], -1
    )
    ix = lax.broadcasted_iota(jnp.int32, (_NCLS, _PADW), 1)
    tv, ti = _bitonic_topk(v, ix, _TOP, _TOP)
    tops_ref[...] = tv
    topi_ref[...] = ti


@jax.jit
def _topk512(logits_t):
    return pl.pallas_call(
        _topk_body,
        out_shape=[
            jax.ShapeDtypeStruct((_NCLS, _TOP), jnp.float32),
            jax.ShapeDtypeStruct((_NCLS, _TOP), jnp.int32),
        ],
    )(logits_t)


_FINW = 8192


def _final_body(s_ref, fs_ref, fi_ref):
    s = s_ref[...]
    v = jnp.concatenate(
        [s, jnp.full((1, _FINW - _NCLS * _TOP), -2.0, jnp.float32)], -1
    )
    ix = lax.broadcasted_iota(jnp.int32, (1, _FINW), 1)
    tv, ti = _bitonic_topk(v, ix, _TOP, _DET)
    fs_ref[...] = tv
    fi_ref[...] = ti


@jax.jit
def _final_topk(scores_cat):
    return pl.pallas_call(
        _final_body,
        out_shape=[
            jax.ShapeDtypeStruct((1, _DET), jnp.float32),
            jax.ShapeDtypeStruct((1, _DET), jnp.int32),
        ],
    )(scores_cat.reshape(1, _NCLS * _TOP))


def _iou_pack_body(props_ref, reg_ref, tops_ref, boxes_ref, sup_ref, valid_ref):
    props = props_ref[0]
    reg = reg_ref[0]
    ctr = props[:, 0:3]
    size = jnp.abs(props[:, 3:6]) + 1e-3
    theta = props[:, 6:7]
    pred_ctr = (reg[:, 0:3] / 10.0) * size + ctr
    pred_size = jnp.exp(jnp.minimum(reg[:, 3:6] / 5.0, _CLIP)) * size
    pred_theta = theta + reg[:, 6:7]
    boxes = jnp.concatenate(
        [pred_ctr, pred_size, pred_theta, jnp.zeros((_TOP, 1), jnp.float32)],
        axis=-1,
    )
    boxes_ref[0] = boxes

    x1 = boxes[:, 0] - boxes[:, 3] * 0.5
    x2 = boxes[:, 0] + boxes[:, 3] * 0.5
    y1 = boxes[:, 1] - boxes[:, 4] * 0.5
    y2 = boxes[:, 1] + boxes[:, 4] * 0.5
    z1 = boxes[:, 2]
    z2 = boxes[:, 2] + boxes[:, 5]

    def inter(a1, a2):
        lo = jnp.maximum(a1[:, None], a1[None, :])
        hi = jnp.minimum(a2[:, None], a2[None, :])
        return jnp.clip(hi - lo, 0.0)

    iv = inter(x1, x2) * inter(y1, y2) * inter(z1, z2)
    vol = (
        jnp.clip(x2 - x1, 0.0)
        * jnp.clip(y2 - y1, 0.0)
        * jnp.clip(z2 - z1, 0.0)
    )
    union = vol[:, None] + vol[None, :] - iv
    iou = iv / jnp.maximum(union, 1e-8)
    col = lax.broadcasted_iota(jnp.int32, (_TOP, _TOP), 1)
    row = lax.broadcasted_iota(jnp.int32, (_TOP, _TOP), 0)
    sup = ((iou > _NMS_THRESH) & (col > row)).astype(jnp.int32)
    weights = jnp.left_shift(
        jnp.int32(1), jnp.arange(32, dtype=jnp.int32)
    )
    words = [
        jnp.sum(sup[:, 32 * w : 32 * w + 32] * weights[None, :], axis=-1)
        for w in range(_W)
    ]
    sup_ref[0] = jnp.concatenate([x[:, None] for x in words], axis=-1)

    vbits = (tops_ref[0, 0] > _SCORE_THRESH).astype(jnp.int32)
    vwords = [
        jnp.sum(vbits[32 * w : 32 * w + 32] * weights) for w in range(_W)
    ]
    valid_ref[0, 0] = jnp.stack(vwords)


@jax.jit
def _iou_pack(props, reg, top_s):
    return pl.pallas_call(
        _iou_pack_body,
        grid=(_NCLS,),
        in_specs=[
            pl.BlockSpec((1, _TOP, 7), lambda c: (c, 0, 0)),
            pl.BlockSpec((1, _TOP, 7), lambda c: (c, 0, 0)),
            pl.BlockSpec((1, 1, _TOP), lambda c: (c, 0, 0)),
        ],
        out_specs=[
            pl.BlockSpec((1, _TOP, 8), lambda c: (c, 0, 0)),
            pl.BlockSpec((1, _TOP, _W), lambda c: (c, 0, 0)),
            pl.BlockSpec((1, 1, _W), lambda c: (c, 0, 0)),
        ],
        out_shape=[
            jax.ShapeDtypeStruct((_NCLS, _TOP, 8), jnp.float32),
            jax.ShapeDtypeStruct((_NCLS, _TOP, _W), jnp.int32),
            jax.ShapeDtypeStruct((_NCLS, 1, _W), jnp.int32),
        ],
    )(props, reg, top_s.reshape(_NCLS, 1, _TOP))


def _pack_bits(bits):
    shape = bits.shape[:-1] + (bits.shape[-1] // 32, 32)
    weights = jnp.left_shift(
        jnp.uint32(1), jnp.arange(32, dtype=jnp.uint32)
    )
    words = jnp.sum(bits.reshape(shape).astype(jnp.uint32) * weights, axis=-1)
    return lax.bitcast_convert_type(words, jnp.int32)


def kernel(class_logits, box_regression, corners_semantic, proposals):
    top_s, top_i = _topk512(class_logits.T)

    props = proposals[top_i]
    reg_all = box_regression.reshape(_N, _C, 7)
    cls_idx = jnp.arange(1, _C, dtype=jnp.int32)[:, None]
    reg = reg_all[top_i, cls_idx]

    boxes8, sup_words3, valid_words3 = _iou_pack(props, reg, top_s)
    sup_words = sup_words3.reshape(_NCLS, _TOP * _W)
    valid_words = valid_words3.reshape(_NCLS, _W)

    keep_words = _run_nms(sup_words, valid_words)
    keep = (
        jnp.right_shift(
            lax.bitcast_convert_type(keep_words, jnp.uint32)[:, :, None],
            jnp.arange(32, dtype=jnp.uint32)[None, None, :],
        )
        & 1
    ).astype(bool).reshape(_NCLS, _TOP)

    s_final = jnp.where(keep, top_s, -1.0)
    fs, fi = _final_topk(s_final)
    final_s = fs[0]
    final_idx = fi[0]
    boxes_cat = boxes8.reshape(-1, 8)
    final_b = boxes_cat[final_idx][:, :7]
    final_l = (final_idx // _TOP + 1).astype(jnp.int32)
    return final_b, final_s, final_l


# speedup vs baseline: 1.1630x; 1.1576x over previous
import functools

import jax
import jax.numpy as jnp
import numpy as np
from jax import lax
from jax.experimental import pallas as pl
from jax.experimental.pallas import tpu as pltpu
from jax.experimental.pallas import tpu_sc as plsc

_N = 20000
_C = 10
_NCLS = _C - 1
_SCORE_THRESH = 0.05
_NMS_THRESH = 0.5
_DET = 100
_TOP = 512
_CLIP = float(np.log(1000.0 / 16.0))
_W = _TOP // 32


def _nms_sc_body(sup_hbm, valid_hbm, out_hbm, sup_v, keep_v):
    nc = 2
    wid = lax.axis_index("s") * nc + lax.axis_index("c")

    @pl.when(wid < _NCLS)
    def _():
        pltpu.sync_copy(sup_hbm.at[wid], sup_v)
        pltpu.sync_copy(valid_hbm.at[wid], keep_v)

        dnums = lax.GatherDimensionNumbers(
            offset_dims=(), collapsed_slice_dims=(0,), start_index_map=(0,)
        )

        def body(i, keep):
            w = lax.shift_right_logical(i, 5)
            b = lax.bitwise_and(i, 31)
            w_vec = jnp.full((16,), w, jnp.int32)
            word = lax.gather(
                keep,
                w_vec[:, None],
                dimension_numbers=dnums,
                slice_sizes=(1,),
                mode=lax.GatherScatterMode.PROMISE_IN_BOUNDS,
            )
            b_vec = jnp.full((16,), b, jnp.int32)
            bit = lax.bitwise_and(lax.shift_right_logical(word, b_vec), 1)
            mask = jnp.where(bit == 1, -1, 0).astype(jnp.int32)
            row = sup_v[pl.ds(i * _W, _W)]
            return lax.bitwise_and(
                keep, lax.bitwise_not(lax.bitwise_and(row, mask))
            )

        keep_v[:] = lax.fori_loop(0, _TOP, body, keep_v[:])
        pltpu.sync_copy(keep_v, out_hbm.at[wid])


@jax.jit
def _run_nms(sup_words, valid_words):
    mesh = plsc.VectorSubcoreMesh(core_axis_name="c", subcore_axis_name="s")
    f = pl.kernel(
        _nms_sc_body,
        out_type=jax.ShapeDtypeStruct((_NCLS, _W), jnp.int32),
        scratch_types=[
            pltpu.VMEM((_TOP * _W,), jnp.int32),
            pltpu.VMEM((_W,), jnp.int32),
        ],
        mesh=mesh,
    )
    return f(sup_words, valid_words)


def _cmpex(v, ix, j, up, ivec):
    left = (ivec & j) == 0
    pvp = jnp.concatenate([v[:, j:], v[:, :j]], -1)
    pvm = jnp.concatenate([v[:, -j:], v[:, :-j]], -1)
    pip = jnp.concatenate([ix[:, j:], ix[:, :j]], -1)
    pim = jnp.concatenate([ix[:, -j:], ix[:, :-j]], -1)
    pv = jnp.where(left, pvp, pvm)
    pi = jnp.where(left, pip, pim)
    a_first = (v > pv) | ((v == pv) & (ix < pi))
    keep = a_first ^ up ^ left
    return jnp.where(keep, v, pv), jnp.where(keep, ix, pi)


def _bitonic_sort_rows(v, ix, alt_by_row):
    bsz, width = v.shape
    ivec = lax.broadcasted_iota(jnp.int32, (bsz, width), 1)
    rpar = (lax.broadcasted_iota(jnp.int32, (bsz, width), 0) & 1) == 1
    k = 2
    while k <= width:
        j = k // 2
        while j >= 1:
            up = (ivec & k) == 0
            if alt_by_row:
                up = up ^ rpar
            v, ix = _cmpex(v, ix, j, up, ivec)
            j //= 2
        k *= 2
    return v, ix


def _bitonic_merge(v, ix, ch, k_out):
    bsz, width = v.shape

    def iota(w):
        return lax.broadcasted_iota(jnp.int32, (bsz, w), 1)

    while width > ch:
        nl = width // ch
        if nl % 2 == 1:
            v = jnp.concatenate(
                [v, jnp.full((bsz, ch), -3.0, v.dtype)], -1
            )
            ix = jnp.concatenate(
                [ix, jnp.full((bsz, ch), jnp.int32(2**30)), ], -1
            )
            width += ch
            nl += 1
        ivec = iota(width)
        j = ch
        while j >= 1:
            v, ix = _cmpex(v, ix, j, (ivec & (2 * ch)) == 0, ivec)
            j //= 2
        offs = [b * 2 * ch + (0 if b % 2 == 0 else ch) for b in range(nl // 2)]
        v = jnp.concatenate([v[:, o : o + ch] for o in offs], -1)
        ix = jnp.concatenate([ix[:, o : o + ch] for o in offs], -1)
        width //= 2
    return v[:, :k_out], ix[:, :k_out]


_PADW = 20480


def _softmax_body(logits_ref, sm_ref):
    x = logits_ref[...]
    m = jnp.max(x, axis=0, keepdims=True)
    e = jnp.exp(x - m)
    denom = jnp.sum(e, axis=0, keepdims=True)
    s = e[1:, :] / denom
    sm = jnp.where(s > _SCORE_THRESH, s, -1.0)
    sm_ref[...] = jnp.concatenate(
        [sm, jnp.full((_NCLS, _PADW - _N), -2.0, jnp.float32)], -1
    )


@jax.jit
def _softmax_pad(logits_t):
    return pl.pallas_call(
        _softmax_body,
        out_shape=jax.ShapeDtypeStruct((_NCLS, _PADW), jnp.float32),
    )(logits_t)


_CHROWS = _PADW // _TOP


def _sortrows_body(v_ref, sv_ref, si_ref):
    v = v_ref[...]
    rr = lax.broadcasted_iota(jnp.int32, v.shape, 0)
    ll = lax.broadcasted_iota(jnp.int32, v.shape, 1)
    ix = (rr % _CHROWS) * _TOP + ll
    sv, si = _bitonic_sort_rows(v, ix, alt_by_row=True)
    sv_ref[...] = sv
    si_ref[...] = si


@jax.jit
def _sortrows(v):
    return pl.pallas_call(
        _sortrows_body,
        out_shape=[
            jax.ShapeDtypeStruct(v.shape, jnp.float32),
            jax.ShapeDtypeStruct(v.shape, jnp.int32),
        ],
    )(v)


def _merge512_body(v_ref, i_ref, tops_ref, topi_ref):
    tv, ti = _bitonic_merge(v_ref[...], i_ref[...], _TOP, _TOP)
    tops_ref[...] = tv
    topi_ref[...] = ti


@jax.jit
def _merge512(v, ix):
    return pl.pallas_call(
        _merge512_body,
        out_shape=[
            jax.ShapeDtypeStruct((_NCLS, _TOP), jnp.float32),
            jax.ShapeDtypeStruct((_NCLS, _TOP), jnp.int32),
        ],
    )(v, ix)


_FINW = 8192


def _postnms_sort_body(tops_ref, keep_ref, sv_ref, si_ref):
    top_s = tops_ref[...]
    words = keep_ref[...]
    ll = lax.broadcasted_iota(jnp.int32, (_NCLS, _TOP), 1)
    wsel = ll >> 5
    wordvec = jnp.zeros((_NCLS, _TOP), jnp.int32)
    for w in range(_W):
        wordvec = jnp.where(wsel == w, words[:, w][:, None], wordvec)
    bit = (wordvec >> (ll & 31)) & 1
    s_final = jnp.where(bit == 1, top_s, -1.0)
    rr = lax.broadcasted_iota(jnp.int32, (_NCLS, _TOP), 0)
    ix = rr * _TOP + ll
    sv, si = _bitonic_sort_rows(s_final, ix, alt_by_row=True)
    sv_ref[...] = sv
    si_ref[...] = si


@jax.jit
def _postnms_sort(top_s, keep_words):
    return pl.pallas_call(
        _postnms_sort_body,
        out_shape=[
            jax.ShapeDtypeStruct((_NCLS, _TOP), jnp.float32),
            jax.ShapeDtypeStruct((_NCLS, _TOP), jnp.int32),
        ],
    )(top_s, keep_words)


def _final_body(s_ref, i_ref, fs_ref, fi_ref):
    v = jnp.concatenate(
        [s_ref[...], jnp.full((1, _FINW - _NCLS * _TOP), -2.0, jnp.float32)],
        -1,
    )
    ix = jnp.concatenate(
        [i_ref[...], jnp.full((1, _FINW - _NCLS * _TOP), jnp.int32(2**30))],
        -1,
    )
    tv, ti = _bitonic_merge(v, ix, _TOP, _DET)
    fs_ref[...] = tv
    fi_ref[...] = ti


@jax.jit
def _final_topk(sv, si):
    return pl.pallas_call(
        _final_body,
        out_shape=[
            jax.ShapeDtypeStruct((1, _DET), jnp.float32),
            jax.ShapeDtypeStruct((1, _DET), jnp.int32),
        ],
    )(sv.reshape(1, _NCLS * _TOP), si.reshape(1, _NCLS * _TOP))


def _iou_pack_body(props_ref, reg_ref, tops_ref, boxes_ref, sup_ref, valid_ref):
    props = props_ref[0]
    reg = reg_ref[0]
    ctr = props[:, 0:3]
    size = jnp.abs(props[:, 3:6]) + 1e-3
    theta = props[:, 6:7]
    pred_ctr = (reg[:, 0:3] / 10.0) * size + ctr
    pred_size = jnp.exp(jnp.minimum(reg[:, 3:6] / 5.0, _CLIP)) * size
    pred_theta = theta + reg[:, 6:7]
    boxes = jnp.concatenate(
        [pred_ctr, pred_size, pred_theta, jnp.zeros((_TOP, 1), jnp.float32)],
        axis=-1,
    )
    boxes_ref[0] = boxes

    x1 = boxes[:, 0] - boxes[:, 3] * 0.5
    x2 = boxes[:, 0] + boxes[:, 3] * 0.5
    y1 = boxes[:, 1] - boxes[:, 4] * 0.5
    y2 = boxes[:, 1] + boxes[:, 4] * 0.5
    z1 = boxes[:, 2]
    z2 = boxes[:, 2] + boxes[:, 5]

    def inter(a1, a2):
        lo = jnp.maximum(a1[:, None], a1[None, :])
        hi = jnp.minimum(a2[:, None], a2[None, :])
        return jnp.clip(hi - lo, 0.0)

    iv = inter(x1, x2) * inter(y1, y2) * inter(z1, z2)
    vol = (
        jnp.clip(x2 - x1, 0.0)
        * jnp.clip(y2 - y1, 0.0)
        * jnp.clip(z2 - z1, 0.0)
    )
    union = vol[:, None] + vol[None, :] - iv
    iou = iv / jnp.maximum(union, 1e-8)
    col = lax.broadcasted_iota(jnp.int32, (_TOP, _TOP), 1)
    row = lax.broadcasted_iota(jnp.int32, (_TOP, _TOP), 0)
    sup = ((iou > _NMS_THRESH) & (col > row)).astype(jnp.int32)
    weights = jnp.left_shift(
        jnp.int32(1), jnp.arange(32, dtype=jnp.int32)
    )
    words = [
        jnp.sum(sup[:, 32 * w : 32 * w + 32] * weights[None, :], axis=-1)
        for w in range(_W)
    ]
    sup_ref[0] = jnp.concatenate([x[:, None] for x in words], axis=-1)

    vbits = (tops_ref[0, 0] > _SCORE_THRESH).astype(jnp.int32)
    vwords = [
        jnp.sum(vbits[32 * w : 32 * w + 32] * weights) for w in range(_W)
    ]
    valid_ref[0, 0] = jnp.stack(vwords)


@jax.jit
def _iou_pack(props, reg, top_s):
    return pl.pallas_call(
        _iou_pack_body,
        grid=(_NCLS,),
        in_specs=[
            pl.BlockSpec((1, _TOP, 7), lambda c: (c, 0, 0)),
            pl.BlockSpec((1, _TOP, 7), lambda c: (c, 0, 0)),
            pl.BlockSpec((1, 1, _TOP), lambda c: (c, 0, 0)),
        ],
        out_specs=[
            pl.BlockSpec((1, _TOP, 8), lambda c: (c, 0, 0)),
            pl.BlockSpec((1, _TOP, _W), lambda c: (c, 0, 0)),
            pl.BlockSpec((1, 1, _W), lambda c: (c, 0, 0)),
        ],
        out_shape=[
            jax.ShapeDtypeStruct((_NCLS, _TOP, 8), jnp.float32),
            jax.ShapeDtypeStruct((_NCLS, _TOP, _W), jnp.int32),
            jax.ShapeDtypeStruct((_NCLS, 1, _W), jnp.int32),
        ],
    )(props, reg, top_s.reshape(_NCLS, 1, _TOP))


def _pack_bits(bits):
    shape = bits.shape[:-1] + (bits.shape[-1] // 32, 32)
    weights = jnp.left_shift(
        jnp.uint32(1), jnp.arange(32, dtype=jnp.uint32)
    )
    words = jnp.sum(bits.reshape(shape).astype(jnp.uint32) * weights, axis=-1)
    return lax.bitcast_convert_type(words, jnp.int32)


def kernel(class_logits, box_regression, corners_semantic, proposals):
    sm = _softmax_pad(class_logits.T)
    cv, ci = _sortrows(sm.reshape(_NCLS * _CHROWS, _TOP))
    top_s, top_i = _merge512(
        cv.reshape(_NCLS, _PADW), ci.reshape(_NCLS, _PADW)
    )

    props = proposals[top_i]
    reg_all = box_regression.reshape(_N, _C, 7)
    cls_idx = jnp.arange(1, _C, dtype=jnp.int32)[:, None]
    reg = reg_all[top_i, cls_idx]

    boxes8, sup_words3, valid_words3 = _iou_pack(props, reg, top_s)
    sup_words = sup_words3.reshape(_NCLS, _TOP * _W)
    valid_words = valid_words3.reshape(_NCLS, _W)

    keep_words = _run_nms(sup_words, valid_words)
    sv, si = _postnms_sort(top_s, keep_words)
    fs, fi = _final_topk(sv, si)
    final_s = fs[0]
    final_idx = fi[0]
    boxes_cat = boxes8.reshape(-1, 8)
    final_b = boxes_cat[final_idx][:, :7]
    final_l = (final_idx // _TOP + 1).astype(jnp.int32)
    return final_b, final_s, final_l
